# Initial kernel scaffold; baseline (speedup 1.0000x reference)
import functools

import jax
import jax.numpy as jnp
from jax import lax
from jax.experimental import pallas as pl
from jax.experimental.pallas import tpu as pltpu
from jax.experimental.pallas import tpu_sc as plsc

WORD_DIM = 50
POS_DIM = 5
OUT_DIM = 60

_INFO = plsc.get_sparse_core_info()
_NC = _INFO.num_cores
_NS = _INFO.num_subcores
_NW = _NC * _NS


@functools.partial(jax.jit, static_argnums=(6, 7))
def _emb(word_i, pos1_i, pos2_i, word_table, pos1_table, pos2_table, n_tok, chunk):
    n_per_w = n_tok // _NW
    n_chunks = n_per_w // chunk
    mesh = plsc.VectorSubcoreMesh(core_axis_name="c", subcore_axis_name="s")

    @functools.partial(
        pl.kernel,
        mesh=mesh,
        out_type=jax.ShapeDtypeStruct((n_tok, OUT_DIM), jnp.float32),
        scratch_types=[
            pltpu.VMEM((chunk,), jnp.int32),
            pltpu.VMEM((chunk,), jnp.int32),
            pltpu.VMEM((chunk,), jnp.int32),
            pltpu.VMEM((chunk, WORD_DIM), jnp.float32),
            pltpu.VMEM((chunk, POS_DIM), jnp.float32),
            pltpu.VMEM((chunk, POS_DIM), jnp.float32),
            pltpu.SemaphoreType.DMA,
        ],
    )
    def body(wi_hbm, p1i_hbm, p2i_hbm, wt_hbm, p1t_hbm, p2t_hbm, out_hbm,
             wi_v, p1i_v, p2i_v, w_v, p1_v, p2_v, sem):
        wid = lax.axis_index("s") * _NC + lax.axis_index("c")
        w_base = wid * n_per_w

        def chunk_body(i):
            base = w_base + i * chunk
            pltpu.sync_copy(wi_hbm.at[pl.ds(base, chunk)], wi_v)
            pltpu.sync_copy(p1i_hbm.at[pl.ds(base, chunk)], p1i_v)
            pltpu.sync_copy(p2i_hbm.at[pl.ds(base, chunk)], p2i_v)
            cw = pltpu.async_copy(wt_hbm.at[wi_v], w_v, sem)
            c1 = pltpu.async_copy(p1t_hbm.at[p1i_v], p1_v, sem)
            c2 = pltpu.async_copy(p2t_hbm.at[p2i_v], p2_v, sem)
            cw.wait()
            c1.wait()
            c2.wait()
            pltpu.sync_copy(w_v, out_hbm.at[pl.ds(base, chunk), pl.ds(0, WORD_DIM)])
            pltpu.sync_copy(p1_v, out_hbm.at[pl.ds(base, chunk), pl.ds(WORD_DIM, POS_DIM)])
            pltpu.sync_copy(p2_v, out_hbm.at[pl.ds(base, chunk), pl.ds(WORD_DIM + POS_DIM, POS_DIM)])

        pl.loop(0, n_chunks)(chunk_body)

    return body(word_i, pos1_i, pos2_i, word_table, pos1_table, pos2_table)


def kernel(word, pos1, pos2, word_table, pos1_table, pos2_table):
    b, l = word.shape
    n = b * l
    wi = word.reshape(n).astype(jnp.int32)
    p1i = pos1.reshape(n).astype(jnp.int32)
    p2i = pos2.reshape(n).astype(jnp.int32)
    out = _emb(wi, p1i, p2i, word_table, pos1_table, pos2_table, n, 1024)
    return out.reshape(b, l, OUT_DIM)


# speedup vs baseline: 5.2054x; 5.2054x over previous
import functools

import jax
import jax.numpy as jnp
from jax import lax
from jax.experimental import pallas as pl
from jax.experimental.pallas import tpu as pltpu
from jax.experimental.pallas import tpu_sc as plsc

WORD_DIM = 50
POS_DIM = 5
POS_ROWS = 400
OUT_DIM = 60
WPAD = 128
PT_LEN = 2 * POS_ROWS * POS_DIM

_INFO = plsc.get_sparse_core_info()
_NC = _INFO.num_cores
_NS = _INFO.num_subcores
_NW = _NC * _NS


@functools.partial(jax.jit, static_argnums=(5, 6))
def _emb(word_i, pos1_i, pos2_i, word_table_pad, pos_cat, n_tok, chunk):
    n_per_w = n_tok // _NW
    n_chunks = n_per_w // chunk
    n_vec = chunk // 16
    mesh = plsc.VectorSubcoreMesh(core_axis_name="c", subcore_axis_name="s")

    @functools.partial(
        pl.kernel,
        mesh=mesh,
        out_type=jax.ShapeDtypeStruct((n_tok * OUT_DIM,), jnp.float32),
        compiler_params=pltpu.CompilerParams(needs_layout_passes=False),
        scratch_types=[
            pltpu.VMEM((chunk,), jnp.int32),
            pltpu.VMEM((chunk,), jnp.int32),
            pltpu.VMEM((chunk,), jnp.int32),
            pltpu.VMEM((chunk, WPAD), jnp.float32),
            pltpu.VMEM((chunk * OUT_DIM,), jnp.float32),
            pltpu.VMEM((PT_LEN,), jnp.float32),
            pltpu.SemaphoreType.DMA,
        ],
    )
    def body(wi_hbm, p1i_hbm, p2i_hbm, wt_hbm, pt_hbm, out_hbm,
             wi_v, p1i_v, p2i_v, w_v, out_v, pt_v, sem):
        wid = lax.axis_index("s") * _NC + lax.axis_index("c")
        w_base = wid * n_per_w
        pltpu.sync_copy(pt_hbm, pt_v)

        def chunk_body(i):
            base = w_base + i * chunk
            pltpu.sync_copy(wi_hbm.at[pl.ds(base, chunk)], wi_v)
            pltpu.sync_copy(p1i_hbm.at[pl.ds(base, chunk)], p1i_v)
            pltpu.sync_copy(p2i_hbm.at[pl.ds(base, chunk)], p2i_v)
            pltpu.async_copy(wt_hbm.at[wi_v], w_v, sem).wait()

            def tok_body(t):
                obase = t * OUT_DIM
                for f32_off in (0, 16, 32, 44):
                    out_v[pl.ds(obase + f32_off, 16)] = w_v[t, pl.ds(f32_off, 16)]

            pl.loop(0, chunk, unroll=8)(tok_body)

            iota = lax.iota(jnp.int32, 16)

            def vec_body(j):
                obase = j * (16 * OUT_DIM) + iota * OUT_DIM + WORD_DIM
                p1 = p1i_v[pl.ds(j * 16, 16)] * POS_DIM
                p2 = p2i_v[pl.ds(j * 16, 16)] * POS_DIM + (POS_ROWS * POS_DIM)
                for d in range(POS_DIM):
                    v1 = plsc.load_gather(pt_v, [p1 + d])
                    plsc.store_scatter(out_v, [obase + d], v1)
                    v2 = plsc.load_gather(pt_v, [p2 + d])
                    plsc.store_scatter(out_v, [obase + POS_DIM + d], v2)

            pl.loop(0, n_vec)(vec_body)

            pltpu.sync_copy(out_v, out_hbm.at[pl.ds(base * OUT_DIM, chunk * OUT_DIM)])

        pl.loop(0, n_chunks)(chunk_body)

    return body(word_i, pos1_i, pos2_i, word_table_pad, pos_cat)


def kernel(word, pos1, pos2, word_table, pos1_table, pos2_table):
    b, l = word.shape
    n = b * l
    wi = word.reshape(n).astype(jnp.int32)
    p1i = pos1.reshape(n).astype(jnp.int32)
    p2i = pos2.reshape(n).astype(jnp.int32)
    wt_pad = jnp.pad(word_table, ((0, 0), (0, WPAD - WORD_DIM)))
    pos_cat = jnp.concatenate(
        [pos1_table.reshape(-1), pos2_table.reshape(-1)])
    out = _emb(wi, p1i, p2i, wt_pad, pos_cat, n, 512)
    return out.reshape(b, l, OUT_DIM)


# speedup vs baseline: 6.1164x; 1.1750x over previous
import functools

import jax
import jax.numpy as jnp
from jax import lax
from jax.experimental import pallas as pl
from jax.experimental.pallas import tpu as pltpu
from jax.experimental.pallas import tpu_sc as plsc

WORD_DIM = 50
POS_DIM = 5
POS_ROWS = 400
OUT_DIM = 60
WPAD = 128
PT_LEN = 2 * POS_ROWS * POS_DIM

_INFO = plsc.get_sparse_core_info()
_NC = _INFO.num_cores
_NS = _INFO.num_subcores
_NW = _NC * _NS


@functools.partial(jax.jit, static_argnums=(5, 6))
def _emb(word_i, pos1_i, pos2_i, word_table_pad, pos_cat, n_tok, chunk):
    n_per_w = n_tok // _NW
    n_chunks = n_per_w // chunk
    assert n_chunks % 2 == 0 and n_chunks >= 4
    n_vec = chunk // 16
    mesh = plsc.VectorSubcoreMesh(core_axis_name="c", subcore_axis_name="s")

    @functools.partial(
        pl.kernel,
        mesh=mesh,
        out_type=jax.ShapeDtypeStruct((n_tok * OUT_DIM,), jnp.float32),
        compiler_params=pltpu.CompilerParams(needs_layout_passes=False),
        scratch_types=[
            [pltpu.VMEM((chunk,), jnp.int32)] * 2,
            [pltpu.VMEM((chunk,), jnp.int32)] * 2,
            [pltpu.VMEM((chunk,), jnp.int32)] * 2,
            [pltpu.VMEM((chunk, WPAD), jnp.float32)] * 2,
            [pltpu.VMEM((chunk * OUT_DIM,), jnp.float32)] * 2,
            pltpu.VMEM((PT_LEN,), jnp.float32),
            [pltpu.SemaphoreType.DMA] * 2,
            [pltpu.SemaphoreType.DMA] * 2,
            [pltpu.SemaphoreType.DMA] * 2,
        ],
    )
    def body(wi_hbm, p1i_hbm, p2i_hbm, wt_hbm, pt_hbm, out_hbm,
             wi_v, p1i_v, p2i_v, w_v, out_v, pt_v, sem_i, sem_g, sem_o):
        wid = lax.axis_index("s") * _NC + lax.axis_index("c")
        w_base = wid * n_per_w
        pltpu.sync_copy(pt_hbm, pt_v)

        def start_idx(g, b):
            base = w_base + g * chunk
            pltpu.async_copy(wi_hbm.at[pl.ds(base, chunk)], wi_v[b], sem_i[b])
            pltpu.async_copy(p1i_hbm.at[pl.ds(base, chunk)], p1i_v[b], sem_i[b])
            pltpu.async_copy(p2i_hbm.at[pl.ds(base, chunk)], p2i_v[b], sem_i[b])

        def wait_idx(b):
            pltpu.make_async_copy(wi_hbm.at[pl.ds(0, chunk)], wi_v[b], sem_i[b]).wait()
            pltpu.make_async_copy(p1i_hbm.at[pl.ds(0, chunk)], p1i_v[b], sem_i[b]).wait()
            pltpu.make_async_copy(p2i_hbm.at[pl.ds(0, chunk)], p2i_v[b], sem_i[b]).wait()

        def start_gather(b):
            pltpu.async_copy(wt_hbm.at[wi_v[b]], w_v[b], sem_g[b])

        def wait_gather(b):
            pltpu.make_async_copy(wt_hbm.at[pl.ds(0, chunk)], w_v[b], sem_g[b]).wait()

        def start_out(g, b):
            base = w_base + g * chunk
            pltpu.async_copy(
                out_v[b], out_hbm.at[pl.ds(base * OUT_DIM, chunk * OUT_DIM)],
                sem_o[b])

        def wait_out(b):
            pltpu.make_async_copy(
                out_v[b], out_hbm.at[pl.ds(0, chunk * OUT_DIM)], sem_o[b]).wait()

        def compute(b):
            def tok_body(t):
                obase = t * OUT_DIM
                for f32_off in (0, 16, 32, 44):
                    out_v[b][pl.ds(obase + f32_off, 16)] = w_v[b][t, pl.ds(f32_off, 16)]

            pl.loop(0, chunk, unroll=8)(tok_body)

            iota = lax.iota(jnp.int32, 16)

            def vec_body(j):
                obase = j * (16 * OUT_DIM) + iota * OUT_DIM + WORD_DIM
                p1 = p1i_v[b][pl.ds(j * 16, 16)] * POS_DIM
                p2 = p2i_v[b][pl.ds(j * 16, 16)] * POS_DIM + (POS_ROWS * POS_DIM)
                for d in range(POS_DIM):
                    v1 = plsc.load_gather(pt_v, [p1 + d])
                    plsc.store_scatter(out_v[b], [obase + d], v1)
                    v2 = plsc.load_gather(pt_v, [p2 + d])
                    plsc.store_scatter(out_v[b], [obase + POS_DIM + d], v2)

            pl.loop(0, n_vec)(vec_body)

        start_idx(0, 0)
        wait_idx(0)
        start_gather(0)
        start_idx(1, 1)

        def outer(g0):
            for par in range(2):
                g = g0 + par
                b = par
                nb = 1 - par
                wait_gather(b)

                @pl.when(g + 1 < n_chunks)
                def _():
                    wait_idx(nb)
                    start_gather(nb)

                @pl.when(g >= 2)
                def _():
                    wait_out(b)

                compute(b)
                start_out(g, b)

                @pl.when(g + 2 < n_chunks)
                def _():
                    start_idx(g + 2, b)

        pl.loop(0, n_chunks, step=2)(outer)
        wait_out(0)
        wait_out(1)

    return body(word_i, pos1_i, pos2_i, word_table_pad, pos_cat)


def kernel(word, pos1, pos2, word_table, pos1_table, pos2_table):
    b, l = word.shape
    n = b * l
    wi = word.reshape(n).astype(jnp.int32)
    p1i = pos1.reshape(n).astype(jnp.int32)
    p2i = pos2.reshape(n).astype(jnp.int32)
    wt_pad = jnp.pad(word_table, ((0, 0), (0, WPAD - WORD_DIM)))
    pos_cat = jnp.concatenate(
        [pos1_table.reshape(-1), pos2_table.reshape(-1)])
    out = _emb(wi, p1i, p2i, wt_pad, pos_cat, n, 256)
    return out.reshape(b, l, OUT_DIM)
